# Initial kernel scaffold; baseline (speedup 1.0000x reference)
import jax
import jax.numpy as jnp
from jax.experimental import pallas as pl


def kernel(x, path_index, W, att, bias):
    raise NotImplementedError("write your pallas kernel here")


# speedup vs baseline: 23.1054x; 23.1054x over previous
import functools

import jax
import jax.numpy as jnp
from jax import lax
from jax.experimental import pallas as pl
from jax.experimental.pallas import tpu as pltpu
from jax.experimental.pallas import tpu_sc as plsc

N = 10000
NP = 10240
E = 320000
EMB = 128
HEADS = 4
OUT = 64
HO = HEADS * OUT

EPT = E // 16
CHUNK = 80
NCHUNK = EPT // CHUNK
STRIPE = NP // 16


def _proj_body(x_ref, w_ref, b_ref, ht_ref, scal_ref, c8_ref):
  i = pl.program_id(0)
  p = pl.program_id(1)
  hblk = jnp.dot(x_ref[...], w_ref[...], preferred_element_type=jnp.float32)
  ht_ref[...] = hblk
  part = jnp.dot(hblk, b_ref[...], preferred_element_type=jnp.float32)

  @pl.when(p == 0)
  def _():
    scal_ref[...] = part

  @pl.when(p == 1)
  def _():
    s = scal_ref[...] + part
    scal_ref[...] = s
    m = jnp.max(s, axis=0, keepdims=True)

    @pl.when(i == 0)
    def _():
      c8_ref[...] = m

    @pl.when(i > 0)
    def _():
      c8_ref[...] = jnp.maximum(c8_ref[...], m)


def _project(x, W, B):
  bn = 1000
  return pl.pallas_call(
      _proj_body,
      grid=(N // bn, 2),
      in_specs=[
          pl.BlockSpec((bn, EMB), lambda i, p: (i, 0)),
          pl.BlockSpec((EMB, EMB), lambda i, p: (0, p)),
          pl.BlockSpec((EMB, 8), lambda i, p: (p, 0)),
      ],
      out_specs=[
          pl.BlockSpec((bn, EMB), lambda i, p: (p * (N // bn) + i, 0)),
          pl.BlockSpec((bn, 8), lambda i, p: (i, 0)),
          pl.BlockSpec((1, 8), lambda i, p: (0, 0)),
      ],
      out_shape=[
          jax.ShapeDtypeStruct((2 * N, EMB), jnp.float32),
          jax.ShapeDtypeStruct((N, 8), jnp.float32),
          jax.ShapeDtypeStruct((1, 8), jnp.float32),
      ],
  )(x, W, B)


def _attn_body(scalt_h, c16_h, src_h, dst_h, a_h,
               tblD, tblS, tblDen, srcv, dstv, a_st,
               tmp, acc, c_v, sp_den, sp_comb):
  cid = lax.axis_index("c")
  sid = lax.axis_index("s")
  ebase = sid * EPT
  zero16 = jnp.zeros((16,), jnp.float32)

  pltpu.sync_copy(c16_h, c_v)
  cb = [plsc.load_gather(c_v, [jnp.full((16,), 2 * cid + hd, jnp.int32)])
        for hd in range(2)]
  for hd in range(2):
    pltpu.sync_copy(scalt_h.at[pl.ds((2 * cid + hd) * NP, NP)], tblD.at[hd])
    pltpu.sync_copy(scalt_h.at[pl.ds((4 + 2 * cid + hd) * NP, NP)],
                    tblS.at[hd])

  def zden(j, _):
    tblDen[0, pl.ds(j * 16, 16)] = zero16
    tblDen[1, pl.ds(j * 16, 16)] = zero16
    return 0
  lax.fori_loop(0, NP // 16, zden, 0)

  def p1(k, _):
    e0 = ebase + k * CHUNK
    pltpu.sync_copy(src_h.at[pl.ds(e0, CHUNK)], srcv)
    pltpu.sync_copy(dst_h.at[pl.ds(e0, CHUNK)], dstv)

    def vec(v, _):
      sv = srcv[pl.ds(v * 16, 16)]
      dv = dstv[pl.ds(v * 16, 16)]
      for hd in range(2):
        hs = jnp.full((16,), hd, jnp.int32)
        s = plsc.load_gather(tblD, [hs, dv]) + plsc.load_gather(tblS, [hs, sv])
        al = jnp.where(s > 0, s, 0.2 * s)
        ex = jnp.exp(al - cb[hd])
        plsc.addupdate_scatter(tblDen, [hs, dv], ex)
      return 0
    lax.fori_loop(0, 5, vec, 0)
    return 0
  lax.fori_loop(0, NCHUNK, p1, 0)

  for hd in range(2):
    pltpu.sync_copy(tblDen.at[hd],
                    sp_den.at[pl.ds((sid * 2 + hd) * NP, NP)])
  plsc.subcore_barrier()

  def czero(j, _):
    acc[0, pl.ds(j * 16, 16)] = zero16
    acc[1, pl.ds(j * 16, 16)] = zero16
    return 0
  lax.fori_loop(0, STRIPE // 16, czero, 0)

  def comb(t, _):
    for hd in range(2):
      pltpu.sync_copy(
          sp_den.at[pl.ds((t * 2 + hd) * NP + sid * STRIPE, STRIPE)], tmp)

      def addv(j, _):
        acc[hd, pl.ds(j * 16, 16)] = (acc[hd, pl.ds(j * 16, 16)]
                                      + tmp[pl.ds(j * 16, 16)])
        return 0
      lax.fori_loop(0, STRIPE // 16, addv, 0)
    return 0
  lax.fori_loop(0, 16, comb, 0)

  for hd in range(2):
    pltpu.sync_copy(acc.at[hd],
                    sp_comb.at[pl.ds(hd * NP + sid * STRIPE, STRIPE)])
  plsc.subcore_barrier()
  for hd in range(2):
    pltpu.sync_copy(sp_comb.at[pl.ds(hd * NP, NP)], tblDen.at[hd])

  def p2(k, _):
    e0 = ebase + k * CHUNK
    pltpu.sync_copy(src_h.at[pl.ds(e0, CHUNK)], srcv)
    pltpu.sync_copy(dst_h.at[pl.ds(e0, CHUNK)], dstv)

    def vec(v, _):
      sv = srcv[pl.ds(v * 16, 16)]
      dv = dstv[pl.ds(v * 16, 16)]
      for hd in range(2):
        hs = jnp.full((16,), hd, jnp.int32)
        s = plsc.load_gather(tblD, [hs, dv]) + plsc.load_gather(tblS, [hs, sv])
        al = jnp.where(s > 0, s, 0.2 * s)
        ex = jnp.exp(al - cb[hd])
        den = plsc.load_gather(tblDen, [hs, dv])
        a_st[hd, pl.ds(v * 16, 16)] = ex / (den + 1e-16)
      return 0
    lax.fori_loop(0, 5, vec, 0)

    for hd in range(2):
      pltpu.sync_copy(a_st.at[hd],
                      a_h.at[pl.ds((2 * cid + hd) * E + e0, CHUNK)])
    return 0
  lax.fori_loop(0, NCHUNK, p2, 0)


def _attn_kernel(scalt, c16, src, dst):
  mesh = plsc.VectorSubcoreMesh(core_axis_name="c", subcore_axis_name="s")
  k = pl.kernel(
      _attn_body,
      out_type=jax.ShapeDtypeStruct((HEADS * E,), jnp.float32),
      mesh=mesh,
      compiler_params=pltpu.CompilerParams(needs_layout_passes=False),
      scratch_types=[
          pltpu.VMEM((2, NP), jnp.float32),
          pltpu.VMEM((2, NP), jnp.float32),
          pltpu.VMEM((2, NP), jnp.float32),
          pltpu.VMEM((CHUNK,), jnp.int32),
          pltpu.VMEM((CHUNK,), jnp.int32),
          pltpu.VMEM((2, CHUNK), jnp.float32),
          pltpu.VMEM((STRIPE,), jnp.float32),
          pltpu.VMEM((2, STRIPE), jnp.float32),
          pltpu.VMEM((16,), jnp.float32),
          pltpu.VMEM_SHARED((16 * 2 * NP,), jnp.float32),
          pltpu.VMEM_SHARED((2 * NP,), jnp.float32),
      ],
  )
  return k(scalt, c16, src, dst)


def _msg_body(ht_h, a_h, src_h, dst_h, agg_h,
              rows, srcv, dstv, sadj, a_st, sem, sp_agg):
  cid = lax.axis_index("c")
  sid = lax.axis_index("s")
  ebase = sid * EPT
  zero16 = jnp.zeros((16,), jnp.float32)

  def zrow(i2, _):
    for j in range(8):
      rows[i2, pl.ds(j * 16, 16)] = zero16
    return 0
  lax.fori_loop(0, CHUNK, zrow, 0)

  def zsp(b, _):
    pltpu.sync_copy(rows, sp_agg.at[pl.ds(sid * STRIPE + b * CHUNK, CHUNK)])
    return 0
  lax.fori_loop(0, STRIPE // CHUNK, zsp, 0)
  plsc.subcore_barrier()

  def p3(k, _):
    e0 = ebase + k * CHUNK
    pltpu.sync_copy(src_h.at[pl.ds(e0, CHUNK)], srcv)
    pltpu.sync_copy(dst_h.at[pl.ds(e0, CHUNK)], dstv)
    for hd in range(2):
      pltpu.sync_copy(a_h.at[pl.ds((2 * cid + hd) * E + e0, CHUNK)],
                      a_st.at[hd])

    def vec(v, _):
      sadj[pl.ds(v * 16, 16)] = srcv[pl.ds(v * 16, 16)] + cid * N
      return 0
    lax.fori_loop(0, 5, vec, 0)

    pltpu.async_copy(ht_h.at[sadj], rows, sem).wait()

    def scale(i2, _):
      i2v = jnp.full((16,), i2, jnp.int32)
      for hd in range(2):
        ab = plsc.load_gather(a_st, [jnp.full((16,), hd, jnp.int32), i2v])
        for j in range(4):
          col = hd * 64 + j * 16
          rows[i2, pl.ds(col, 16)] = rows[i2, pl.ds(col, 16)] * ab
      return 0
    lax.fori_loop(0, CHUNK, scale, 0)

    pltpu.sync_copy(rows, sp_agg.at[dstv], add=True)
    return 0
  lax.fori_loop(0, NCHUNK, p3, 0)

  plsc.subcore_barrier()
  pltpu.sync_copy(
      sp_agg.at[pl.ds(sid * STRIPE, STRIPE)],
      agg_h.at[pl.ds(cid * NP + sid * STRIPE, STRIPE)])


def _msg_kernel(ht, a_flat, src, dst):
  mesh = plsc.VectorSubcoreMesh(core_axis_name="c", subcore_axis_name="s")
  k = pl.kernel(
      _msg_body,
      out_type=jax.ShapeDtypeStruct((2 * NP, EMB), jnp.float32),
      mesh=mesh,
      compiler_params=pltpu.CompilerParams(needs_layout_passes=False),
      scratch_types=[
          pltpu.VMEM((CHUNK, EMB), jnp.float32),
          pltpu.VMEM((CHUNK,), jnp.int32),
          pltpu.VMEM((CHUNK,), jnp.int32),
          pltpu.VMEM((CHUNK,), jnp.int32),
          pltpu.VMEM((2, CHUNK), jnp.float32),
          pltpu.SemaphoreType.DMA,
          pltpu.VMEM_SHARED((NP, EMB), jnp.float32),
      ],
  )
  return k(ht, a_flat, src, dst)


def _mean_body(agg_ref, bias_ref, out_ref):
  s = (agg_ref[0, :, 0:OUT] + agg_ref[0, :, OUT:EMB]
       + agg_ref[1, :, 0:OUT] + agg_ref[1, :, OUT:EMB])
  out_ref[...] = s * 0.25 + bias_ref[...]


def _head_mean(agg, bias2d):
  bn = 512
  return pl.pallas_call(
      _mean_body,
      grid=(NP // bn,),
      in_specs=[
          pl.BlockSpec((2, bn, EMB), lambda i: (0, i, 0)),
          pl.BlockSpec((1, OUT), lambda i: (0, 0)),
      ],
      out_specs=pl.BlockSpec((bn, OUT), lambda i: (i, 0)),
      out_shape=jax.ShapeDtypeStruct((N, OUT), jnp.float32),
  )(agg, bias2d)


def kernel(x, path_index, W, att, bias):
  src = path_index[0]
  dst = path_index[1]

  att_d = att[0, :, :OUT]
  att_s = att[0, :, OUT:]
  eye = jnp.eye(HEADS, dtype=jnp.float32)
  Bd = jnp.einsum("ho,hk->hok", att_d, eye).reshape(HO, HEADS)
  Bs = jnp.einsum("ho,hk->hok", att_s, eye).reshape(HO, HEADS)
  B = jnp.concatenate([Bd, Bs], axis=1)

  ht, scal, c8 = _project(x, W, B)

  scalt = jnp.pad(scal, ((0, NP - N), (0, 0))).T.reshape(-1)
  c4 = c8[0, :HEADS] + c8[0, HEADS:]
  c4 = jnp.where(c4 > 0, c4, 0.2 * c4)
  c16 = jnp.pad(c4, (0, 12))

  a_flat = _attn_kernel(scalt, c16, src, dst)
  agg = _msg_kernel(ht, a_flat, src, dst)

  out = _head_mean(agg.reshape(2, NP, EMB), bias.reshape(1, OUT))
  a = a_flat.reshape(HEADS, E).T
  return out, a


# speedup vs baseline: 55.4119x; 2.3982x over previous
import functools

import jax
import jax.numpy as jnp
from jax import lax
from jax.experimental import pallas as pl
from jax.experimental.pallas import tpu as pltpu
from jax.experimental.pallas import tpu_sc as plsc

N = 10000
NP = 10240
E = 320000
EMB = 128
HEADS = 4
OUT = 64
HO = HEADS * OUT

EPT = E // 16
CHUNK = 80
NCHUNK = EPT // CHUNK
STRIPE = NP // 16
ABLK = 2000
NABLK = EPT // ABLK
MBLK = 800
NCHB = MBLK // CHUNK
NMBLK = EPT // MBLK


def _proj_body(x_ref, w_ref, b_ref, ht_ref, scal_ref, c8_ref):
  i = pl.program_id(0)
  p = pl.program_id(1)
  hblk = jnp.dot(x_ref[...], w_ref[...], preferred_element_type=jnp.float32)
  ht_ref[...] = hblk
  part = jnp.dot(hblk, b_ref[...], preferred_element_type=jnp.float32)

  @pl.when(p == 0)
  def _():
    scal_ref[...] = part

  @pl.when(p == 1)
  def _():
    s = scal_ref[...] + part
    scal_ref[...] = s
    m = jnp.max(s, axis=0, keepdims=True)

    @pl.when(i == 0)
    def _():
      c8_ref[...] = m

    @pl.when(i > 0)
    def _():
      c8_ref[...] = jnp.maximum(c8_ref[...], m)


def _project(x, W, B):
  bn = 1000
  return pl.pallas_call(
      _proj_body,
      grid=(N // bn, 2),
      in_specs=[
          pl.BlockSpec((bn, EMB), lambda i, p: (i, 0)),
          pl.BlockSpec((EMB, EMB), lambda i, p: (0, p)),
          pl.BlockSpec((EMB, 8), lambda i, p: (p, 0)),
      ],
      out_specs=[
          pl.BlockSpec((bn, EMB), lambda i, p: (p * (N // bn) + i, 0)),
          pl.BlockSpec((bn, 8), lambda i, p: (i, 0)),
          pl.BlockSpec((1, 8), lambda i, p: (0, 0)),
      ],
      out_shape=[
          jax.ShapeDtypeStruct((2 * N, EMB), jnp.float32),
          jax.ShapeDtypeStruct((N, 8), jnp.float32),
          jax.ShapeDtypeStruct((1, 8), jnp.float32),
      ],
  )(x, W, B)


def _attn_body(scalt_h, c16_h, src_h, dst_h, a_h,
               tblD, tblS, tblDen, srcv, dstv, a_st0, a_st1,
               tmp, acc, c_v, sp_den, sp_comb):
  a_sts = (a_st0, a_st1)
  cid = lax.axis_index("c")
  sid = lax.axis_index("s")
  ebase = sid * EPT
  zero16 = jnp.zeros((16,), jnp.float32)

  pltpu.sync_copy(c16_h, c_v)
  cb = [plsc.load_gather(c_v, [jnp.full((16,), 2 * cid + hd, jnp.int32)])
        for hd in range(2)]
  for hd in range(2):
    pltpu.sync_copy(scalt_h.at[pl.ds((2 * cid + hd) * NP, NP)], tblD.at[hd])
    pltpu.sync_copy(scalt_h.at[pl.ds((4 + 2 * cid + hd) * NP, NP)],
                    tblS.at[hd])

  def zden(j, _):
    tblDen[0, pl.ds(j * 16, 16)] = zero16
    tblDen[1, pl.ds(j * 16, 16)] = zero16
    return 0
  lax.fori_loop(0, NP // 16, zden, 0)

  def p1(k, _):
    e0 = ebase + k * ABLK
    pltpu.sync_copy(src_h.at[pl.ds(e0, ABLK)], srcv)
    pltpu.sync_copy(dst_h.at[pl.ds(e0, ABLK)], dstv)

    def vec(v, _):
      sv = srcv[pl.ds(v * 16, 16)]
      dv = dstv[pl.ds(v * 16, 16)]
      for hd in range(2):
        hs = jnp.full((16,), hd, jnp.int32)
        s = plsc.load_gather(tblD, [hs, dv]) + plsc.load_gather(tblS, [hs, sv])
        al = jnp.where(s > 0, s, 0.2 * s)
        ex = jnp.exp(al - cb[hd])
        plsc.addupdate_scatter(tblDen, [hs, dv], ex)
      return 0
    lax.fori_loop(0, ABLK // 16, vec, 0)
    return 0
  lax.fori_loop(0, NABLK, p1, 0)

  for hd in range(2):
    pltpu.sync_copy(tblDen.at[hd],
                    sp_den.at[pl.ds((sid * 2 + hd) * NP, NP)])
  plsc.subcore_barrier()

  def czero(j, _):
    acc[0, pl.ds(j * 16, 16)] = zero16
    acc[1, pl.ds(j * 16, 16)] = zero16
    return 0
  lax.fori_loop(0, STRIPE // 16, czero, 0)

  def comb(t, _):
    for hd in range(2):
      pltpu.sync_copy(
          sp_den.at[pl.ds((t * 2 + hd) * NP + sid * STRIPE, STRIPE)], tmp)

      def addv(j, _):
        acc[hd, pl.ds(j * 16, 16)] = (acc[hd, pl.ds(j * 16, 16)]
                                      + tmp[pl.ds(j * 16, 16)])
        return 0
      lax.fori_loop(0, STRIPE // 16, addv, 0)
    return 0
  lax.fori_loop(0, 16, comb, 0)

  for hd in range(2):
    pltpu.sync_copy(acc.at[hd],
                    sp_comb.at[pl.ds(hd * NP + sid * STRIPE, STRIPE)])
  plsc.subcore_barrier()
  for hd in range(2):
    pltpu.sync_copy(sp_comb.at[pl.ds(hd * NP, NP)], tblDen.at[hd])

  def p2(k, _):
    e0 = ebase + k * ABLK
    pltpu.sync_copy(src_h.at[pl.ds(e0, ABLK)], srcv)
    pltpu.sync_copy(dst_h.at[pl.ds(e0, ABLK)], dstv)

    def vec(v, _):
      sv = srcv[pl.ds(v * 16, 16)]
      dv = dstv[pl.ds(v * 16, 16)]
      for hd in range(2):
        hs = jnp.full((16,), hd, jnp.int32)
        s = plsc.load_gather(tblD, [hs, dv]) + plsc.load_gather(tblS, [hs, sv])
        al = jnp.where(s > 0, s, 0.2 * s)
        ex = jnp.exp(al - cb[hd])
        den = plsc.load_gather(tblDen, [hs, dv])
        a_sts[hd][pl.ds(v * 16, 16)] = ex / (den + 1e-16)
      return 0
    lax.fori_loop(0, ABLK // 16, vec, 0)

    for hd in range(2):
      pltpu.sync_copy(a_sts[hd],
                      a_h.at[pl.ds((2 * cid + hd) * E + e0, ABLK)])
    return 0
  lax.fori_loop(0, NABLK, p2, 0)


def _attn_kernel(scalt, c16, src, dst):
  mesh = plsc.VectorSubcoreMesh(core_axis_name="c", subcore_axis_name="s")
  k = pl.kernel(
      _attn_body,
      out_type=jax.ShapeDtypeStruct((HEADS * E,), jnp.float32),
      mesh=mesh,
      compiler_params=pltpu.CompilerParams(needs_layout_passes=False),
      scratch_types=[
          pltpu.VMEM((2, NP), jnp.float32),
          pltpu.VMEM((2, NP), jnp.float32),
          pltpu.VMEM((2, NP), jnp.float32),
          pltpu.VMEM((ABLK,), jnp.int32),
          pltpu.VMEM((ABLK,), jnp.int32),
          pltpu.VMEM((ABLK,), jnp.float32),
          pltpu.VMEM((ABLK,), jnp.float32),
          pltpu.VMEM((STRIPE,), jnp.float32),
          pltpu.VMEM((2, STRIPE), jnp.float32),
          pltpu.VMEM((16,), jnp.float32),
          pltpu.VMEM_SHARED((16 * 2 * NP,), jnp.float32),
          pltpu.VMEM_SHARED((2 * NP,), jnp.float32),
      ],
  )
  return k(scalt, c16, src, dst)


def _msg_body(ht_h, a_h, src_h, dst_h, agg_h,
              rows0, rows1, out0, out1, srcv, sadj, dstv2, a_st0, a_st1,
              gsem0, gsem1, ssem0, ssem1, sp_agg):
  a_sts = (a_st0, a_st1)
  rows = (rows0, rows1)
  out = (out0, out1)
  gsem = (gsem0, gsem1)
  ssem = (ssem0, ssem1)
  cid = lax.axis_index("c")
  sid = lax.axis_index("s")
  ebase = sid * EPT
  zero16 = jnp.zeros((16,), jnp.float32)

  def zrow(i2, _):
    for j in range(8):
      rows0[i2, pl.ds(j * 16, 16)] = zero16
    return 0
  lax.fori_loop(0, CHUNK, zrow, 0)

  def zsp(b, _):
    pltpu.sync_copy(rows0, sp_agg.at[pl.ds(sid * STRIPE + b * CHUNK, CHUNK)])
    return 0
  lax.fori_loop(0, STRIPE // CHUNK, zsp, 0)
  plsc.subcore_barrier()

  def blk(kb, _):
    e0 = ebase + kb * MBLK
    pltpu.sync_copy(src_h.at[pl.ds(e0, MBLK)], srcv)
    for jj in range(NCHB):
      pltpu.sync_copy(dst_h.at[pl.ds(e0 + jj * CHUNK, CHUNK)], dstv2.at[jj])
    for hd in range(2):
      pltpu.sync_copy(a_h.at[pl.ds((2 * cid + hd) * E + e0, MBLK)],
                      a_sts[hd])

    def vec(v, _):
      sadj[pl.ds(v * 16, 16)] = srcv[pl.ds(v * 16, 16)] + cid * N
      return 0
    lax.fori_loop(0, MBLK // 16, vec, 0)

    pltpu.async_copy(ht_h.at[sadj.at[pl.ds(0, CHUNK)]], rows0, gsem0)
    pltpu.async_copy(ht_h.at[sadj.at[pl.ds(CHUNK, CHUNK)]], rows1, gsem1)

    def pair(j2, _):
      for b in range(2):
        j = 2 * j2 + b
        pltpu.make_async_copy(
            ht_h.at[sadj.at[pl.ds(j * CHUNK, CHUNK)]], rows[b],
            gsem[b]).wait()

        @pl.when(j2 >= 1)
        def _():
          pltpu.make_async_copy(out[b], sp_agg.at[dstv2.at[j - 2]],
                                ssem[b]).wait()

        def scale(i2, _):
          i2v = jnp.full((16,), j * CHUNK + i2, jnp.int32)
          for hd in range(2):
            ab = plsc.load_gather(a_sts[hd], [i2v])
            for j4 in range(4):
              col = hd * 64 + j4 * 16
              out[b][i2, pl.ds(col, 16)] = rows[b][i2, pl.ds(col, 16)] * ab
          return 0
        lax.fori_loop(0, CHUNK, scale, 0)

        pltpu.async_copy(out[b], sp_agg.at[dstv2.at[j]], ssem[b], add=True)

        @pl.when(j + 2 < NCHB)
        def _():
          pltpu.async_copy(ht_h.at[sadj.at[pl.ds((j + 2) * CHUNK, CHUNK)]],
                           rows[b], gsem[b])
      return 0
    lax.fori_loop(0, NCHB // 2, pair, 0)

    for b in range(2):
      pltpu.make_async_copy(out[b], sp_agg.at[dstv2.at[NCHB - 2 + b]],
                            ssem[b]).wait()
    return 0
  lax.fori_loop(0, NMBLK, blk, 0)

  plsc.subcore_barrier()
  pltpu.sync_copy(
      sp_agg.at[pl.ds(sid * STRIPE, STRIPE)],
      agg_h.at[pl.ds(cid * NP + sid * STRIPE, STRIPE)])


def _msg_kernel(ht, a_flat, src, dst):
  mesh = plsc.VectorSubcoreMesh(core_axis_name="c", subcore_axis_name="s")
  k = pl.kernel(
      _msg_body,
      out_type=jax.ShapeDtypeStruct((2 * NP, EMB), jnp.float32),
      mesh=mesh,
      compiler_params=pltpu.CompilerParams(needs_layout_passes=False),
      scratch_types=[
          pltpu.VMEM((CHUNK, EMB), jnp.float32),
          pltpu.VMEM((CHUNK, EMB), jnp.float32),
          pltpu.VMEM((CHUNK, EMB), jnp.float32),
          pltpu.VMEM((CHUNK, EMB), jnp.float32),
          pltpu.VMEM((MBLK,), jnp.int32),
          pltpu.VMEM((MBLK,), jnp.int32),
          pltpu.VMEM((NCHB, CHUNK), jnp.int32),
          pltpu.VMEM((MBLK,), jnp.float32),
          pltpu.VMEM((MBLK,), jnp.float32),
          pltpu.SemaphoreType.DMA,
          pltpu.SemaphoreType.DMA,
          pltpu.SemaphoreType.DMA,
          pltpu.SemaphoreType.DMA,
          pltpu.VMEM_SHARED((NP, EMB), jnp.float32),
      ],
  )
  return k(ht, a_flat, src, dst)


def _mean_body(agg_ref, bias_ref, out_ref):
  s = (agg_ref[0, :, 0:OUT] + agg_ref[0, :, OUT:EMB]
       + agg_ref[1, :, 0:OUT] + agg_ref[1, :, OUT:EMB])
  out_ref[...] = s * 0.25 + bias_ref[...]


def _head_mean(agg, bias2d):
  bn = 512
  return pl.pallas_call(
      _mean_body,
      grid=(NP // bn,),
      in_specs=[
          pl.BlockSpec((2, bn, EMB), lambda i: (0, i, 0)),
          pl.BlockSpec((1, OUT), lambda i: (0, 0)),
      ],
      out_specs=pl.BlockSpec((bn, OUT), lambda i: (i, 0)),
      out_shape=jax.ShapeDtypeStruct((N, OUT), jnp.float32),
  )(agg, bias2d)


def kernel(x, path_index, W, att, bias):
  src = path_index[0]
  dst = path_index[1]

  att_d = att[0, :, :OUT]
  att_s = att[0, :, OUT:]
  eye = jnp.eye(HEADS, dtype=jnp.float32)
  Bd = jnp.einsum("ho,hk->hok", att_d, eye).reshape(HO, HEADS)
  Bs = jnp.einsum("ho,hk->hok", att_s, eye).reshape(HO, HEADS)
  B = jnp.concatenate([Bd, Bs], axis=1)

  ht, scal, c8 = _project(x, W, B)

  scalt = jnp.pad(scal, ((0, NP - N), (0, 0))).T.reshape(-1)
  c4 = c8[0, :HEADS] + c8[0, HEADS:]
  c4 = jnp.where(c4 > 0, c4, 0.2 * c4)
  c16 = jnp.pad(c4, (0, 12))

  a_flat = _attn_kernel(scalt, c16, src, dst)
  agg = _msg_kernel(ht, a_flat, src, dst)

  out = _head_mean(agg.reshape(2, NP, EMB), bias.reshape(1, OUT))
  a = a_flat.reshape(HEADS, E).T
  return out, a


# speedup vs baseline: 71.3474x; 1.2876x over previous
import functools

import jax
import jax.numpy as jnp
from jax import lax
from jax.experimental import pallas as pl
from jax.experimental.pallas import tpu as pltpu
from jax.experimental.pallas import tpu_sc as plsc

N = 10000
NP = 10240
E = 320000
EMB = 128
HEADS = 4
OUT = 64
HO = HEADS * OUT

EPT = E // 16
CHUNK = 80
NCHUNK = EPT // CHUNK
STRIPE = NP // 16
ABLK = 2000
NABLK = EPT // ABLK
MBLK = 800
NCHB = MBLK // CHUNK
NMBLK = EPT // MBLK


def _proj_body(x_ref, w_ref, b_ref, ht_ref, scal_ref, c8_ref):
  i = pl.program_id(0)
  p = pl.program_id(1)
  hblk = jnp.dot(x_ref[...], w_ref[...], preferred_element_type=jnp.float32)
  ht_ref[...] = hblk
  part = jnp.dot(hblk, b_ref[...], preferred_element_type=jnp.float32)

  @pl.when(p == 0)
  def _():
    scal_ref[...] = part

  @pl.when(p == 1)
  def _():
    s = scal_ref[...] + part
    scal_ref[...] = s
    m = jnp.max(s, axis=0, keepdims=True)

    @pl.when(i == 0)
    def _():
      c8_ref[...] = m

    @pl.when(i > 0)
    def _():
      c8_ref[...] = jnp.maximum(c8_ref[...], m)


def _project(x, W, B):
  bn = 1000
  return pl.pallas_call(
      _proj_body,
      grid=(N // bn, 2),
      in_specs=[
          pl.BlockSpec((bn, EMB), lambda i, p: (i, 0)),
          pl.BlockSpec((EMB, EMB), lambda i, p: (0, p)),
          pl.BlockSpec((EMB, 8), lambda i, p: (p, 0)),
      ],
      out_specs=[
          pl.BlockSpec((bn, EMB), lambda i, p: (p * (N // bn) + i, 0)),
          pl.BlockSpec((bn, 8), lambda i, p: (i, 0)),
          pl.BlockSpec((1, 8), lambda i, p: (0, 0)),
      ],
      out_shape=[
          jax.ShapeDtypeStruct((2 * N, EMB), jnp.float32),
          jax.ShapeDtypeStruct((N, 8), jnp.float32),
          jax.ShapeDtypeStruct((1, 8), jnp.float32),
      ],
  )(x, W, B)


def _attn_body(scalt_h, c16_h, src_h, dst_h, a_h,
               tblD, tblS, tblDen, srcv, dstv, a_st0, a_st1,
               tmp, acc, c_v, sp_den, sp_comb):
  a_sts = (a_st0, a_st1)
  cid = lax.axis_index("c")
  sid = lax.axis_index("s")
  ebase = sid * EPT
  zero16 = jnp.zeros((16,), jnp.float32)

  pltpu.sync_copy(c16_h, c_v)
  cb = [plsc.load_gather(c_v, [jnp.full((16,), 2 * cid + hd, jnp.int32)])
        for hd in range(2)]
  for hd in range(2):
    pltpu.sync_copy(scalt_h.at[pl.ds((2 * cid + hd) * NP, NP)], tblD.at[hd])
    pltpu.sync_copy(scalt_h.at[pl.ds((4 + 2 * cid + hd) * NP, NP)],
                    tblS.at[hd])

  def zden(j, _):
    tblDen[0, pl.ds(j * 16, 16)] = zero16
    tblDen[1, pl.ds(j * 16, 16)] = zero16
    return 0
  lax.fori_loop(0, NP // 16, zden, 0)

  def p1(k, _):
    e0 = ebase + k * ABLK
    pltpu.sync_copy(src_h.at[pl.ds(e0, ABLK)], srcv)
    pltpu.sync_copy(dst_h.at[pl.ds(e0, ABLK)], dstv)

    @plsc.parallel_loop(0, ABLK // 16, unroll=5)
    def vec(v):
      sv = srcv[pl.ds(v * 16, 16)]
      dv = dstv[pl.ds(v * 16, 16)]
      for hd in range(2):
        hs = jnp.full((16,), hd, jnp.int32)
        s = plsc.load_gather(tblD, [hs, dv]) + plsc.load_gather(tblS, [hs, sv])
        al = jnp.where(s > 0, s, 0.2 * s)
        ex = jnp.exp(al - cb[hd])
        plsc.addupdate_scatter(tblDen, [hs, dv], ex)
    return 0
  lax.fori_loop(0, NABLK, p1, 0)

  for hd in range(2):
    pltpu.sync_copy(tblDen.at[hd],
                    sp_den.at[pl.ds((sid * 2 + hd) * NP, NP)])
  plsc.subcore_barrier()

  def czero(j, _):
    acc[0, pl.ds(j * 16, 16)] = zero16
    acc[1, pl.ds(j * 16, 16)] = zero16
    return 0
  lax.fori_loop(0, STRIPE // 16, czero, 0)

  def comb(t, _):
    for hd in range(2):
      pltpu.sync_copy(
          sp_den.at[pl.ds((t * 2 + hd) * NP + sid * STRIPE, STRIPE)], tmp)

      def addv(j, _):
        acc[hd, pl.ds(j * 16, 16)] = (acc[hd, pl.ds(j * 16, 16)]
                                      + tmp[pl.ds(j * 16, 16)])
        return 0
      lax.fori_loop(0, STRIPE // 16, addv, 0)
    return 0
  lax.fori_loop(0, 16, comb, 0)

  for hd in range(2):
    pltpu.sync_copy(acc.at[hd],
                    sp_comb.at[pl.ds(hd * NP + sid * STRIPE, STRIPE)])
  plsc.subcore_barrier()
  for hd in range(2):
    pltpu.sync_copy(sp_comb.at[pl.ds(hd * NP, NP)], tblDen.at[hd])

  def p2(k, _):
    e0 = ebase + k * ABLK
    pltpu.sync_copy(src_h.at[pl.ds(e0, ABLK)], srcv)
    pltpu.sync_copy(dst_h.at[pl.ds(e0, ABLK)], dstv)

    @plsc.parallel_loop(0, ABLK // 16, unroll=5)
    def vec(v):
      sv = srcv[pl.ds(v * 16, 16)]
      dv = dstv[pl.ds(v * 16, 16)]
      for hd in range(2):
        hs = jnp.full((16,), hd, jnp.int32)
        s = plsc.load_gather(tblD, [hs, dv]) + plsc.load_gather(tblS, [hs, sv])
        al = jnp.where(s > 0, s, 0.2 * s)
        ex = jnp.exp(al - cb[hd])
        den = plsc.load_gather(tblDen, [hs, dv])
        a_sts[hd][pl.ds(v * 16, 16)] = ex / (den + 1e-16)

    for hd in range(2):
      pltpu.sync_copy(a_sts[hd],
                      a_h.at[pl.ds((2 * cid + hd) * E + e0, ABLK)])
    return 0
  lax.fori_loop(0, NABLK, p2, 0)


def _attn_kernel(scalt, c16, src, dst):
  mesh = plsc.VectorSubcoreMesh(core_axis_name="c", subcore_axis_name="s")
  k = pl.kernel(
      _attn_body,
      out_type=jax.ShapeDtypeStruct((HEADS * E,), jnp.float32),
      mesh=mesh,
      compiler_params=pltpu.CompilerParams(needs_layout_passes=False),
      scratch_types=[
          pltpu.VMEM((2, NP), jnp.float32),
          pltpu.VMEM((2, NP), jnp.float32),
          pltpu.VMEM((2, NP), jnp.float32),
          pltpu.VMEM((ABLK,), jnp.int32),
          pltpu.VMEM((ABLK,), jnp.int32),
          pltpu.VMEM((ABLK,), jnp.float32),
          pltpu.VMEM((ABLK,), jnp.float32),
          pltpu.VMEM((STRIPE,), jnp.float32),
          pltpu.VMEM((2, STRIPE), jnp.float32),
          pltpu.VMEM((16,), jnp.float32),
          pltpu.VMEM_SHARED((16 * 2 * NP,), jnp.float32),
          pltpu.VMEM_SHARED((2 * NP,), jnp.float32),
      ],
  )
  return k(scalt, c16, src, dst)


def _msg_body(ht_h, a_h, src_h, dst_h, agg_h,
              rows0, rows1, out0, out1, srcv, sadj, dstv2, a_st0, a_st1,
              gsem0, gsem1, ssem0, ssem1, sp_agg):
  a_sts = (a_st0, a_st1)
  rows = (rows0, rows1)
  out = (out0, out1)
  gsem = (gsem0, gsem1)
  ssem = (ssem0, ssem1)
  cid = lax.axis_index("c")
  sid = lax.axis_index("s")
  ebase = sid * EPT
  zero16 = jnp.zeros((16,), jnp.float32)

  def zrow(i2, _):
    for j in range(8):
      rows0[i2, pl.ds(j * 16, 16)] = zero16
    return 0
  lax.fori_loop(0, CHUNK, zrow, 0)

  def zsp(b, _):
    pltpu.sync_copy(rows0, sp_agg.at[pl.ds(sid * STRIPE + b * CHUNK, CHUNK)])
    return 0
  lax.fori_loop(0, STRIPE // CHUNK, zsp, 0)
  plsc.subcore_barrier()

  def blk(kb, _):
    e0 = ebase + kb * MBLK
    pltpu.sync_copy(src_h.at[pl.ds(e0, MBLK)], srcv)
    for jj in range(NCHB):
      pltpu.sync_copy(dst_h.at[pl.ds(e0 + jj * CHUNK, CHUNK)], dstv2.at[jj])
    for hd in range(2):
      pltpu.sync_copy(a_h.at[pl.ds((2 * cid + hd) * E + e0, MBLK)],
                      a_sts[hd])

    @plsc.parallel_loop(0, MBLK // 16, unroll=5)
    def vec(v):
      sadj[pl.ds(v * 16, 16)] = srcv[pl.ds(v * 16, 16)] + cid * N

    pltpu.async_copy(ht_h.at[sadj.at[pl.ds(0, CHUNK)]], rows0, gsem0)
    pltpu.async_copy(ht_h.at[sadj.at[pl.ds(CHUNK, CHUNK)]], rows1, gsem1)

    def pair(j2, _):
      for b in range(2):
        j = 2 * j2 + b
        pltpu.make_async_copy(
            ht_h.at[sadj.at[pl.ds(j * CHUNK, CHUNK)]], rows[b],
            gsem[b]).wait()

        @pl.when(j2 >= 1)
        def _():
          pltpu.make_async_copy(out[b], sp_agg.at[dstv2.at[j - 2]],
                                ssem[b]).wait()

        @plsc.parallel_loop(0, CHUNK, unroll=8)
        def scale(i2):
          i2v = jnp.full((16,), j * CHUNK + i2, jnp.int32)
          for hd in range(2):
            ab = plsc.load_gather(a_sts[hd], [i2v])
            for j4 in range(4):
              col = hd * 64 + j4 * 16
              out[b][i2, pl.ds(col, 16)] = rows[b][i2, pl.ds(col, 16)] * ab

        pltpu.async_copy(out[b], sp_agg.at[dstv2.at[j]], ssem[b], add=True)

        @pl.when(j + 2 < NCHB)
        def _():
          pltpu.async_copy(ht_h.at[sadj.at[pl.ds((j + 2) * CHUNK, CHUNK)]],
                           rows[b], gsem[b])
      return 0
    lax.fori_loop(0, NCHB // 2, pair, 0)

    for b in range(2):
      pltpu.make_async_copy(out[b], sp_agg.at[dstv2.at[NCHB - 2 + b]],
                            ssem[b]).wait()
    return 0
  lax.fori_loop(0, NMBLK, blk, 0)

  plsc.subcore_barrier()
  pltpu.sync_copy(
      sp_agg.at[pl.ds(sid * STRIPE, STRIPE)],
      agg_h.at[pl.ds(cid * NP + sid * STRIPE, STRIPE)])


def _msg_kernel(ht, a_flat, src, dst):
  mesh = plsc.VectorSubcoreMesh(core_axis_name="c", subcore_axis_name="s")
  k = pl.kernel(
      _msg_body,
      out_type=jax.ShapeDtypeStruct((2 * NP, EMB), jnp.float32),
      mesh=mesh,
      compiler_params=pltpu.CompilerParams(needs_layout_passes=False),
      scratch_types=[
          pltpu.VMEM((CHUNK, EMB), jnp.float32),
          pltpu.VMEM((CHUNK, EMB), jnp.float32),
          pltpu.VMEM((CHUNK, EMB), jnp.float32),
          pltpu.VMEM((CHUNK, EMB), jnp.float32),
          pltpu.VMEM((MBLK,), jnp.int32),
          pltpu.VMEM((MBLK,), jnp.int32),
          pltpu.VMEM((NCHB, CHUNK), jnp.int32),
          pltpu.VMEM((MBLK,), jnp.float32),
          pltpu.VMEM((MBLK,), jnp.float32),
          pltpu.SemaphoreType.DMA,
          pltpu.SemaphoreType.DMA,
          pltpu.SemaphoreType.DMA,
          pltpu.SemaphoreType.DMA,
          pltpu.VMEM_SHARED((NP, EMB), jnp.float32),
      ],
  )
  return k(ht, a_flat, src, dst)


def _mean_body(agg_ref, bias_ref, out_ref):
  s = (agg_ref[0, :, 0:OUT] + agg_ref[0, :, OUT:EMB]
       + agg_ref[1, :, 0:OUT] + agg_ref[1, :, OUT:EMB])
  out_ref[...] = s * 0.25 + bias_ref[...]


def _head_mean(agg, bias2d):
  bn = 512
  return pl.pallas_call(
      _mean_body,
      grid=(NP // bn,),
      in_specs=[
          pl.BlockSpec((2, bn, EMB), lambda i: (0, i, 0)),
          pl.BlockSpec((1, OUT), lambda i: (0, 0)),
      ],
      out_specs=pl.BlockSpec((bn, OUT), lambda i: (i, 0)),
      out_shape=jax.ShapeDtypeStruct((N, OUT), jnp.float32),
  )(agg, bias2d)


def kernel(x, path_index, W, att, bias):
  src = path_index[0]
  dst = path_index[1]

  att_d = att[0, :, :OUT]
  att_s = att[0, :, OUT:]
  eye = jnp.eye(HEADS, dtype=jnp.float32)
  Bd = jnp.einsum("ho,hk->hok", att_d, eye).reshape(HO, HEADS)
  Bs = jnp.einsum("ho,hk->hok", att_s, eye).reshape(HO, HEADS)
  B = jnp.concatenate([Bd, Bs], axis=1)

  ht, scal, c8 = _project(x, W, B)

  scalt = jnp.pad(scal, ((0, NP - N), (0, 0))).T.reshape(-1)
  c4 = c8[0, :HEADS] + c8[0, HEADS:]
  c4 = jnp.where(c4 > 0, c4, 0.2 * c4)
  c16 = jnp.pad(c4, (0, 12))

  a_flat = _attn_kernel(scalt, c16, src, dst)
  agg = _msg_kernel(ht, a_flat, src, dst)

  out = _head_mean(agg.reshape(2, NP, EMB), bias.reshape(1, OUT))
  a = a_flat.reshape(HEADS, E).T
  return out, a


# speedup vs baseline: 94.0319x; 1.3179x over previous
import functools

import jax
import jax.numpy as jnp
from jax import lax
from jax.experimental import pallas as pl
from jax.experimental.pallas import tpu as pltpu
from jax.experimental.pallas import tpu_sc as plsc

N = 10000
NP = 10240
E = 320000
EMB = 128
HEADS = 4
OUT = 64
HO = HEADS * OUT

EPT = E // 16
CHUNK = 80
NCHUNK = EPT // CHUNK
STRIPE = NP // 16
ABLK = 2000
NABLK = EPT // ABLK
MBLK = 800
NCHB = MBLK // CHUNK
NMBLK = EPT // MBLK


def _proj_body(x_ref, w_ref, b_ref, ht_ref, scal_ref, c8_ref):
  i = pl.program_id(0)
  p = pl.program_id(1)
  hblk = jnp.dot(x_ref[...], w_ref[...], preferred_element_type=jnp.float32)
  ht_ref[...] = hblk
  part = jnp.dot(hblk, b_ref[...], preferred_element_type=jnp.float32)

  @pl.when(p == 0)
  def _():
    scal_ref[...] = part

  @pl.when(p == 1)
  def _():
    s = scal_ref[...] + part
    scal_ref[...] = s
    m = jnp.max(s, axis=0, keepdims=True)

    @pl.when(i == 0)
    def _():
      c8_ref[...] = m

    @pl.when(i > 0)
    def _():
      c8_ref[...] = jnp.maximum(c8_ref[...], m)


def _project(x, W, B):
  bn = 1000
  return pl.pallas_call(
      _proj_body,
      grid=(N // bn, 2),
      in_specs=[
          pl.BlockSpec((bn, EMB), lambda i, p: (i, 0)),
          pl.BlockSpec((EMB, EMB), lambda i, p: (0, p)),
          pl.BlockSpec((EMB, 8), lambda i, p: (p, 0)),
      ],
      out_specs=[
          pl.BlockSpec((bn, EMB), lambda i, p: (p * (N // bn) + i, 0)),
          pl.BlockSpec((bn, 8), lambda i, p: (i, 0)),
          pl.BlockSpec((1, 8), lambda i, p: (0, 0)),
      ],
      out_shape=[
          jax.ShapeDtypeStruct((2 * N, EMB), jnp.float32),
          jax.ShapeDtypeStruct((N, 8), jnp.float32),
          jax.ShapeDtypeStruct((1, 8), jnp.float32),
      ],
  )(x, W, B)


def _attn_body(scalt_h, c16_h, src_h, dst_h, a_h,
               tblD, tblS, tblDen, srcv, dstv, a_st0, a_st1,
               tmp, acc, c_v, bsem, sp_den, sp_comb):
  a_sts = (a_st0, a_st1)
  cid = lax.axis_index("c")
  sid = lax.axis_index("s")
  ebase = sid * EPT
  zero16 = jnp.zeros((16,), jnp.float32)

  pltpu.sync_copy(c16_h, c_v)
  cb = [plsc.load_gather(c_v, [jnp.full((16,), 2 * cid + hd, jnp.int32)])
        for hd in range(2)]
  for hd in range(2):
    pltpu.sync_copy(scalt_h.at[pl.ds((2 * cid + hd) * NP, NP)], tblD.at[hd])
    pltpu.sync_copy(scalt_h.at[pl.ds((4 + 2 * cid + hd) * NP, NP)],
                    tblS.at[hd])

  def zden(j, _):
    tblDen[0, pl.ds(j * 16, 16)] = zero16
    tblDen[1, pl.ds(j * 16, 16)] = zero16
    return 0
  lax.fori_loop(0, NP // 16, zden, 0)

  def p1(k, _):
    e0 = ebase + k * ABLK
    descs = [(src_h.at[pl.ds(e0, ABLK)], srcv),
             (dst_h.at[pl.ds(e0, ABLK)], dstv)]
    for sref, dref in descs:
      pltpu.async_copy(sref, dref, bsem)
    for sref, dref in descs:
      pltpu.make_async_copy(sref, dref, bsem).wait()

    @plsc.parallel_loop(0, ABLK // 16, unroll=5)
    def vec(v):
      sv = srcv[pl.ds(v * 16, 16)]
      dv = dstv[pl.ds(v * 16, 16)]
      for hd in range(2):
        hs = jnp.full((16,), hd, jnp.int32)
        s = plsc.load_gather(tblD, [hs, dv]) + plsc.load_gather(tblS, [hs, sv])
        al = jnp.where(s > 0, s, 0.2 * s)
        ex = jnp.exp(al - cb[hd])
        plsc.addupdate_scatter(tblDen, [hs, dv], ex)
    return 0
  lax.fori_loop(0, NABLK, p1, 0)

  for hd in range(2):
    pltpu.sync_copy(tblDen.at[hd],
                    sp_den.at[pl.ds((sid * 2 + hd) * NP, NP)])
  plsc.subcore_barrier()

  def czero(j, _):
    acc[0, pl.ds(j * 16, 16)] = zero16
    acc[1, pl.ds(j * 16, 16)] = zero16
    return 0
  lax.fori_loop(0, STRIPE // 16, czero, 0)

  def comb(t, _):
    for hd in range(2):
      pltpu.sync_copy(
          sp_den.at[pl.ds((t * 2 + hd) * NP + sid * STRIPE, STRIPE)], tmp)

      def addv(j, _):
        acc[hd, pl.ds(j * 16, 16)] = (acc[hd, pl.ds(j * 16, 16)]
                                      + tmp[pl.ds(j * 16, 16)])
        return 0
      lax.fori_loop(0, STRIPE // 16, addv, 0)
    return 0
  lax.fori_loop(0, 16, comb, 0)

  for hd in range(2):
    pltpu.sync_copy(acc.at[hd],
                    sp_comb.at[pl.ds(hd * NP + sid * STRIPE, STRIPE)])
  plsc.subcore_barrier()
  for hd in range(2):
    pltpu.sync_copy(sp_comb.at[pl.ds(hd * NP, NP)], tblDen.at[hd])

  def p2(k, _):
    e0 = ebase + k * ABLK
    descs = [(src_h.at[pl.ds(e0, ABLK)], srcv),
             (dst_h.at[pl.ds(e0, ABLK)], dstv)]
    for sref, dref in descs:
      pltpu.async_copy(sref, dref, bsem)
    for sref, dref in descs:
      pltpu.make_async_copy(sref, dref, bsem).wait()

    @plsc.parallel_loop(0, ABLK // 16, unroll=5)
    def vec(v):
      sv = srcv[pl.ds(v * 16, 16)]
      dv = dstv[pl.ds(v * 16, 16)]
      for hd in range(2):
        hs = jnp.full((16,), hd, jnp.int32)
        s = plsc.load_gather(tblD, [hs, dv]) + plsc.load_gather(tblS, [hs, sv])
        al = jnp.where(s > 0, s, 0.2 * s)
        ex = jnp.exp(al - cb[hd])
        den = plsc.load_gather(tblDen, [hs, dv])
        a_sts[hd][pl.ds(v * 16, 16)] = ex / (den + 1e-16)

    for hd in range(2):
      pltpu.sync_copy(a_sts[hd],
                      a_h.at[pl.ds((2 * cid + hd) * E + e0, ABLK)])
    return 0
  lax.fori_loop(0, NABLK, p2, 0)


def _attn_kernel(scalt, c16, src, dst):
  mesh = plsc.VectorSubcoreMesh(core_axis_name="c", subcore_axis_name="s")
  k = pl.kernel(
      _attn_body,
      out_type=jax.ShapeDtypeStruct((HEADS * E,), jnp.float32),
      mesh=mesh,
      compiler_params=pltpu.CompilerParams(needs_layout_passes=False),
      scratch_types=[
          pltpu.VMEM((2, NP), jnp.float32),
          pltpu.VMEM((2, NP), jnp.float32),
          pltpu.VMEM((2, NP), jnp.float32),
          pltpu.VMEM((ABLK,), jnp.int32),
          pltpu.VMEM((ABLK,), jnp.int32),
          pltpu.VMEM((ABLK,), jnp.float32),
          pltpu.VMEM((ABLK,), jnp.float32),
          pltpu.VMEM((STRIPE,), jnp.float32),
          pltpu.VMEM((2, STRIPE), jnp.float32),
          pltpu.VMEM((16,), jnp.float32),
          pltpu.SemaphoreType.DMA,
          pltpu.VMEM_SHARED((16 * 2 * NP,), jnp.float32),
          pltpu.VMEM_SHARED((2 * NP,), jnp.float32),
      ],
  )
  return k(scalt, c16, src, dst)


def _msg_body(ht_h, a_h, src_h, dst_h, agg_h,
              rows0, rows1, out0, out1, srcv, sadj, dstv2, a_st0, a_st1,
              gsem0, gsem1, ssem0, ssem1, bsem, sp_agg):
  a_sts = (a_st0, a_st1)
  rows = (rows0, rows1)
  out = (out0, out1)
  gsem = (gsem0, gsem1)
  ssem = (ssem0, ssem1)
  cid = lax.axis_index("c")
  sid = lax.axis_index("s")
  ebase = sid * EPT
  zero16 = jnp.zeros((16,), jnp.float32)

  def zrow(i2, _):
    for j in range(8):
      rows0[i2, pl.ds(j * 16, 16)] = zero16
    return 0
  lax.fori_loop(0, CHUNK, zrow, 0)

  def zsp(b, _):
    pltpu.sync_copy(rows0, sp_agg.at[pl.ds(sid * STRIPE + b * CHUNK, CHUNK)])
    return 0
  lax.fori_loop(0, STRIPE // CHUNK, zsp, 0)
  plsc.subcore_barrier()

  def blk(kb, _):
    e0 = ebase + kb * MBLK
    descs = [(src_h.at[pl.ds(e0, MBLK)], srcv)]
    for jj in range(NCHB):
      descs.append((dst_h.at[pl.ds(e0 + jj * CHUNK, CHUNK)], dstv2.at[jj]))
    for hd in range(2):
      descs.append((a_h.at[pl.ds((2 * cid + hd) * E + e0, MBLK)],
                    a_sts[hd]))
    for sref, dref in descs:
      pltpu.async_copy(sref, dref, bsem)
    for sref, dref in descs:
      pltpu.make_async_copy(sref, dref, bsem).wait()

    @plsc.parallel_loop(0, MBLK // 16, unroll=5)
    def vec(v):
      sadj[pl.ds(v * 16, 16)] = srcv[pl.ds(v * 16, 16)] + cid * N

    pltpu.async_copy(ht_h.at[sadj.at[pl.ds(0, CHUNK)]], rows0, gsem0)
    pltpu.async_copy(ht_h.at[sadj.at[pl.ds(CHUNK, CHUNK)]], rows1, gsem1)

    def pair(j2, _):
      for b in range(2):
        j = 2 * j2 + b
        pltpu.make_async_copy(
            ht_h.at[sadj.at[pl.ds(j * CHUNK, CHUNK)]], rows[b],
            gsem[b]).wait()

        @pl.when(j2 >= 1)
        def _():
          pltpu.make_async_copy(out[b], sp_agg.at[dstv2.at[j - 2]],
                                ssem[b]).wait()

        @plsc.parallel_loop(0, CHUNK, unroll=8)
        def scale(i2):
          i2v = jnp.full((16,), j * CHUNK + i2, jnp.int32)
          for hd in range(2):
            ab = plsc.load_gather(a_sts[hd], [i2v])
            for j4 in range(4):
              col = hd * 64 + j4 * 16
              out[b][i2, pl.ds(col, 16)] = rows[b][i2, pl.ds(col, 16)] * ab

        pltpu.async_copy(out[b], sp_agg.at[dstv2.at[j]], ssem[b], add=True)

        @pl.when(j + 2 < NCHB)
        def _():
          pltpu.async_copy(ht_h.at[sadj.at[pl.ds((j + 2) * CHUNK, CHUNK)]],
                           rows[b], gsem[b])
      return 0
    lax.fori_loop(0, NCHB // 2, pair, 0)

    for b in range(2):
      pltpu.make_async_copy(out[b], sp_agg.at[dstv2.at[NCHB - 2 + b]],
                            ssem[b]).wait()
    return 0
  lax.fori_loop(0, NMBLK, blk, 0)

  plsc.subcore_barrier()
  pltpu.sync_copy(
      sp_agg.at[pl.ds(sid * STRIPE, STRIPE)],
      agg_h.at[pl.ds(cid * NP + sid * STRIPE, STRIPE)])


def _msg_kernel(ht, a_flat, src, dst):
  mesh = plsc.VectorSubcoreMesh(core_axis_name="c", subcore_axis_name="s")
  k = pl.kernel(
      _msg_body,
      out_type=jax.ShapeDtypeStruct((2 * NP, EMB), jnp.float32),
      mesh=mesh,
      compiler_params=pltpu.CompilerParams(needs_layout_passes=False),
      scratch_types=[
          pltpu.VMEM((CHUNK, EMB), jnp.float32),
          pltpu.VMEM((CHUNK, EMB), jnp.float32),
          pltpu.VMEM((CHUNK, EMB), jnp.float32),
          pltpu.VMEM((CHUNK, EMB), jnp.float32),
          pltpu.VMEM((MBLK,), jnp.int32),
          pltpu.VMEM((MBLK,), jnp.int32),
          pltpu.VMEM((NCHB, CHUNK), jnp.int32),
          pltpu.VMEM((MBLK,), jnp.float32),
          pltpu.VMEM((MBLK,), jnp.float32),
          pltpu.SemaphoreType.DMA,
          pltpu.SemaphoreType.DMA,
          pltpu.SemaphoreType.DMA,
          pltpu.SemaphoreType.DMA,
          pltpu.SemaphoreType.DMA,
          pltpu.VMEM_SHARED((NP, EMB), jnp.float32),
      ],
  )
  return k(ht, a_flat, src, dst)


def _mean_body(agg_ref, bias_ref, out_ref):
  s = (agg_ref[0, :, 0:OUT] + agg_ref[0, :, OUT:EMB]
       + agg_ref[1, :, 0:OUT] + agg_ref[1, :, OUT:EMB])
  out_ref[...] = s * 0.25 + bias_ref[...]


def _head_mean(agg, bias2d):
  bn = 512
  return pl.pallas_call(
      _mean_body,
      grid=(NP // bn,),
      in_specs=[
          pl.BlockSpec((2, bn, EMB), lambda i: (0, i, 0)),
          pl.BlockSpec((1, OUT), lambda i: (0, 0)),
      ],
      out_specs=pl.BlockSpec((bn, OUT), lambda i: (i, 0)),
      out_shape=jax.ShapeDtypeStruct((N, OUT), jnp.float32),
  )(agg, bias2d)


def kernel(x, path_index, W, att, bias):
  src = path_index[0]
  dst = path_index[1]

  att_d = att[0, :, :OUT]
  att_s = att[0, :, OUT:]
  eye = jnp.eye(HEADS, dtype=jnp.float32)
  Bd = jnp.einsum("ho,hk->hok", att_d, eye).reshape(HO, HEADS)
  Bs = jnp.einsum("ho,hk->hok", att_s, eye).reshape(HO, HEADS)
  B = jnp.concatenate([Bd, Bs], axis=1)

  ht, scal, c8 = _project(x, W, B)

  scalt = jnp.pad(scal, ((0, NP - N), (0, 0))).T.reshape(-1)
  c4 = c8[0, :HEADS] + c8[0, HEADS:]
  c4 = jnp.where(c4 > 0, c4, 0.2 * c4)
  c16 = jnp.pad(c4, (0, 12))

  a_flat = _attn_kernel(scalt, c16, src, dst)
  agg = _msg_kernel(ht, a_flat, src, dst)

  out = _head_mean(agg.reshape(2, NP, EMB), bias.reshape(1, OUT))
  a = a_flat.reshape(HEADS, E).T
  return out, a


# speedup vs baseline: 95.5500x; 1.0161x over previous
import functools

import jax
import jax.numpy as jnp
from jax import lax
from jax.experimental import pallas as pl
from jax.experimental.pallas import tpu as pltpu
from jax.experimental.pallas import tpu_sc as plsc

N = 10000
NP = 10240
E = 320000
EMB = 128
HEADS = 4
OUT = 64
HO = HEADS * OUT

EPT = E // 16
CHUNK = 80
NCHUNK = EPT // CHUNK
STRIPE = NP // 16
ABLK = 2000
NABLK = EPT // ABLK
MBLK = 800
NCHB = MBLK // CHUNK
NMBLK = EPT // MBLK


def _proj_body(x_ref, w_ref, b_ref, ht_ref, scal_ref, c8_ref):
  i = pl.program_id(0)
  bn = x_ref.shape[0]
  hblk = jnp.dot(x_ref[...], w_ref[...], preferred_element_type=jnp.float32)
  ht_ref[...] = hblk.reshape(bn, 2, EMB).transpose(1, 0, 2)
  sc = jnp.dot(hblk, b_ref[...], preferred_element_type=jnp.float32)
  scal_ref[...] = sc
  m = jnp.max(sc, axis=0, keepdims=True)

  @pl.when(i == 0)
  def _():
    c8_ref[...] = m

  @pl.when(i > 0)
  def _():
    c8_ref[...] = jnp.maximum(c8_ref[...], m)


def _project(x, W, B):
  bn = 1000
  return pl.pallas_call(
      _proj_body,
      grid=(N // bn,),
      in_specs=[
          pl.BlockSpec((bn, EMB), lambda i: (i, 0)),
          pl.BlockSpec((EMB, HO), lambda i: (0, 0)),
          pl.BlockSpec((HO, 8), lambda i: (0, 0)),
      ],
      out_specs=[
          pl.BlockSpec((2, bn, EMB), lambda i: (0, i, 0)),
          pl.BlockSpec((bn, 8), lambda i: (i, 0)),
          pl.BlockSpec((1, 8), lambda i: (0, 0)),
      ],
      out_shape=[
          jax.ShapeDtypeStruct((2, N, EMB), jnp.float32),
          jax.ShapeDtypeStruct((N, 8), jnp.float32),
          jax.ShapeDtypeStruct((1, 8), jnp.float32),
      ],
  )(x, W, B)


def _attn_body(scalt_h, c16_h, src_h, dst_h, a_h,
               tblD, tblS, tblDen, srcv, dstv, a_st0, a_st1,
               tmp, acc, c_v, bsem, sp_den, sp_comb):
  a_sts = (a_st0, a_st1)
  cid = lax.axis_index("c")
  sid = lax.axis_index("s")
  ebase = sid * EPT
  zero16 = jnp.zeros((16,), jnp.float32)

  pltpu.sync_copy(c16_h, c_v)
  cb = [plsc.load_gather(c_v, [jnp.full((16,), 2 * cid + hd, jnp.int32)])
        for hd in range(2)]
  for hd in range(2):
    pltpu.sync_copy(scalt_h.at[pl.ds((2 * cid + hd) * NP, NP)], tblD.at[hd])
    pltpu.sync_copy(scalt_h.at[pl.ds((4 + 2 * cid + hd) * NP, NP)],
                    tblS.at[hd])

  def zden(j, _):
    tblDen[0, pl.ds(j * 16, 16)] = zero16
    tblDen[1, pl.ds(j * 16, 16)] = zero16
    return 0
  lax.fori_loop(0, NP // 16, zden, 0)

  def p1(k, _):
    e0 = ebase + k * ABLK
    descs = [(src_h.at[pl.ds(e0, ABLK)], srcv),
             (dst_h.at[pl.ds(e0, ABLK)], dstv)]
    for sref, dref in descs:
      pltpu.async_copy(sref, dref, bsem)
    for sref, dref in descs:
      pltpu.make_async_copy(sref, dref, bsem).wait()

    @plsc.parallel_loop(0, ABLK // 16, unroll=5)
    def vec(v):
      sv = srcv[pl.ds(v * 16, 16)]
      dv = dstv[pl.ds(v * 16, 16)]
      for hd in range(2):
        hs = jnp.full((16,), hd, jnp.int32)
        s = plsc.load_gather(tblD, [hs, dv]) + plsc.load_gather(tblS, [hs, sv])
        al = jnp.where(s > 0, s, 0.2 * s)
        ex = jnp.exp(al - cb[hd])
        plsc.addupdate_scatter(tblDen, [hs, dv], ex)
    return 0
  lax.fori_loop(0, NABLK, p1, 0)

  for hd in range(2):
    pltpu.sync_copy(tblDen.at[hd],
                    sp_den.at[pl.ds((sid * 2 + hd) * NP, NP)])
  plsc.subcore_barrier()

  def czero(j, _):
    acc[0, pl.ds(j * 16, 16)] = zero16
    acc[1, pl.ds(j * 16, 16)] = zero16
    return 0
  lax.fori_loop(0, STRIPE // 16, czero, 0)

  def comb(t, _):
    for hd in range(2):
      pltpu.sync_copy(
          sp_den.at[pl.ds((t * 2 + hd) * NP + sid * STRIPE, STRIPE)], tmp)

      def addv(j, _):
        acc[hd, pl.ds(j * 16, 16)] = (acc[hd, pl.ds(j * 16, 16)]
                                      + tmp[pl.ds(j * 16, 16)])
        return 0
      lax.fori_loop(0, STRIPE // 16, addv, 0)
    return 0
  lax.fori_loop(0, 16, comb, 0)

  for hd in range(2):
    pltpu.sync_copy(acc.at[hd],
                    sp_comb.at[pl.ds(hd * NP + sid * STRIPE, STRIPE)])
  plsc.subcore_barrier()
  for hd in range(2):
    pltpu.sync_copy(sp_comb.at[pl.ds(hd * NP, NP)], tblDen.at[hd])

  def p2(k, _):
    e0 = ebase + k * ABLK
    descs = [(src_h.at[pl.ds(e0, ABLK)], srcv),
             (dst_h.at[pl.ds(e0, ABLK)], dstv)]
    for sref, dref in descs:
      pltpu.async_copy(sref, dref, bsem)
    for sref, dref in descs:
      pltpu.make_async_copy(sref, dref, bsem).wait()

    @plsc.parallel_loop(0, ABLK // 16, unroll=5)
    def vec(v):
      sv = srcv[pl.ds(v * 16, 16)]
      dv = dstv[pl.ds(v * 16, 16)]
      for hd in range(2):
        hs = jnp.full((16,), hd, jnp.int32)
        s = plsc.load_gather(tblD, [hs, dv]) + plsc.load_gather(tblS, [hs, sv])
        al = jnp.where(s > 0, s, 0.2 * s)
        ex = jnp.exp(al - cb[hd])
        den = plsc.load_gather(tblDen, [hs, dv])
        a_sts[hd][pl.ds(v * 16, 16)] = ex / (den + 1e-16)

    for hd in range(2):
      pltpu.sync_copy(a_sts[hd],
                      a_h.at[pl.ds((2 * cid + hd) * E + e0, ABLK)])
    return 0
  lax.fori_loop(0, NABLK, p2, 0)


def _attn_kernel(scalt, c16, src, dst):
  mesh = plsc.VectorSubcoreMesh(core_axis_name="c", subcore_axis_name="s")
  k = pl.kernel(
      _attn_body,
      out_type=jax.ShapeDtypeStruct((HEADS * E,), jnp.float32),
      mesh=mesh,
      compiler_params=pltpu.CompilerParams(needs_layout_passes=False),
      scratch_types=[
          pltpu.VMEM((2, NP), jnp.float32),
          pltpu.VMEM((2, NP), jnp.float32),
          pltpu.VMEM((2, NP), jnp.float32),
          pltpu.VMEM((ABLK,), jnp.int32),
          pltpu.VMEM((ABLK,), jnp.int32),
          pltpu.VMEM((ABLK,), jnp.float32),
          pltpu.VMEM((ABLK,), jnp.float32),
          pltpu.VMEM((STRIPE,), jnp.float32),
          pltpu.VMEM((2, STRIPE), jnp.float32),
          pltpu.VMEM((16,), jnp.float32),
          pltpu.SemaphoreType.DMA,
          pltpu.VMEM_SHARED((16 * 2 * NP,), jnp.float32),
          pltpu.VMEM_SHARED((2 * NP,), jnp.float32),
      ],
  )
  return k(scalt, c16, src, dst)


def _msg_body(ht_h, a_h, src_h, dst_h, agg_h,
              rows0, rows1, out0, out1, srcv, sadj, dstv2, a_st0, a_st1,
              gsem0, gsem1, ssem0, ssem1, bsem, sp_agg):
  a_sts = (a_st0, a_st1)
  rows = (rows0, rows1)
  out = (out0, out1)
  gsem = (gsem0, gsem1)
  ssem = (ssem0, ssem1)
  cid = lax.axis_index("c")
  sid = lax.axis_index("s")
  ebase = sid * EPT
  zero16 = jnp.zeros((16,), jnp.float32)

  def zrow(i2, _):
    for j in range(8):
      rows0[i2, pl.ds(j * 16, 16)] = zero16
    return 0
  lax.fori_loop(0, CHUNK, zrow, 0)

  def zsp(b, _):
    pltpu.sync_copy(rows0, sp_agg.at[pl.ds(sid * STRIPE + b * CHUNK, CHUNK)])
    return 0
  lax.fori_loop(0, STRIPE // CHUNK, zsp, 0)
  plsc.subcore_barrier()

  def blk(kb, _):
    e0 = ebase + kb * MBLK
    descs = [(src_h.at[pl.ds(e0, MBLK)], srcv)]
    for jj in range(NCHB):
      descs.append((dst_h.at[pl.ds(e0 + jj * CHUNK, CHUNK)], dstv2.at[jj]))
    for hd in range(2):
      descs.append((a_h.at[pl.ds((2 * cid + hd) * E + e0, MBLK)],
                    a_sts[hd]))
    for sref, dref in descs:
      pltpu.async_copy(sref, dref, bsem)
    for sref, dref in descs:
      pltpu.make_async_copy(sref, dref, bsem).wait()

    @plsc.parallel_loop(0, MBLK // 16, unroll=5)
    def vec(v):
      sadj[pl.ds(v * 16, 16)] = srcv[pl.ds(v * 16, 16)] + cid * N

    pltpu.async_copy(ht_h.at[sadj.at[pl.ds(0, CHUNK)]], rows0, gsem0)
    pltpu.async_copy(ht_h.at[sadj.at[pl.ds(CHUNK, CHUNK)]], rows1, gsem1)

    def pair(j2, _):
      for b in range(2):
        j = 2 * j2 + b
        pltpu.make_async_copy(
            ht_h.at[sadj.at[pl.ds(j * CHUNK, CHUNK)]], rows[b],
            gsem[b]).wait()

        @pl.when(j2 >= 1)
        def _():
          pltpu.make_async_copy(out[b], sp_agg.at[dstv2.at[j - 2]],
                                ssem[b]).wait()

        @plsc.parallel_loop(0, CHUNK, unroll=8)
        def scale(i2):
          i2v = jnp.full((16,), j * CHUNK + i2, jnp.int32)
          for hd in range(2):
            ab = plsc.load_gather(a_sts[hd], [i2v])
            for j4 in range(4):
              col = hd * 64 + j4 * 16
              out[b][i2, pl.ds(col, 16)] = rows[b][i2, pl.ds(col, 16)] * ab

        pltpu.async_copy(out[b], sp_agg.at[dstv2.at[j]], ssem[b], add=True)

        @pl.when(j + 2 < NCHB)
        def _():
          pltpu.async_copy(ht_h.at[sadj.at[pl.ds((j + 2) * CHUNK, CHUNK)]],
                           rows[b], gsem[b])
      return 0
    lax.fori_loop(0, NCHB // 2, pair, 0)

    for b in range(2):
      pltpu.make_async_copy(out[b], sp_agg.at[dstv2.at[NCHB - 2 + b]],
                            ssem[b]).wait()
    return 0
  lax.fori_loop(0, NMBLK, blk, 0)

  plsc.subcore_barrier()
  pltpu.sync_copy(
      sp_agg.at[pl.ds(sid * STRIPE, STRIPE)],
      agg_h.at[pl.ds(cid * NP + sid * STRIPE, STRIPE)])


def _msg_kernel(ht, a_flat, src, dst):
  mesh = plsc.VectorSubcoreMesh(core_axis_name="c", subcore_axis_name="s")
  k = pl.kernel(
      _msg_body,
      out_type=jax.ShapeDtypeStruct((2 * NP, EMB), jnp.float32),
      mesh=mesh,
      compiler_params=pltpu.CompilerParams(needs_layout_passes=False),
      scratch_types=[
          pltpu.VMEM((CHUNK, EMB), jnp.float32),
          pltpu.VMEM((CHUNK, EMB), jnp.float32),
          pltpu.VMEM((CHUNK, EMB), jnp.float32),
          pltpu.VMEM((CHUNK, EMB), jnp.float32),
          pltpu.VMEM((MBLK,), jnp.int32),
          pltpu.VMEM((MBLK,), jnp.int32),
          pltpu.VMEM((NCHB, CHUNK), jnp.int32),
          pltpu.VMEM((MBLK,), jnp.float32),
          pltpu.VMEM((MBLK,), jnp.float32),
          pltpu.SemaphoreType.DMA,
          pltpu.SemaphoreType.DMA,
          pltpu.SemaphoreType.DMA,
          pltpu.SemaphoreType.DMA,
          pltpu.SemaphoreType.DMA,
          pltpu.VMEM_SHARED((NP, EMB), jnp.float32),
      ],
  )
  return k(ht, a_flat, src, dst)


def _mean_body(agg_ref, bias_ref, out_ref):
  s = (agg_ref[0, :, 0:OUT] + agg_ref[0, :, OUT:EMB]
       + agg_ref[1, :, 0:OUT] + agg_ref[1, :, OUT:EMB])
  out_ref[...] = s * 0.25 + bias_ref[...]


def _head_mean(agg, bias2d):
  bn = 512
  return pl.pallas_call(
      _mean_body,
      grid=(NP // bn,),
      in_specs=[
          pl.BlockSpec((2, bn, EMB), lambda i: (0, i, 0)),
          pl.BlockSpec((1, OUT), lambda i: (0, 0)),
      ],
      out_specs=pl.BlockSpec((bn, OUT), lambda i: (i, 0)),
      out_shape=jax.ShapeDtypeStruct((N, OUT), jnp.float32),
  )(agg, bias2d)


def kernel(x, path_index, W, att, bias):
  src = path_index[0]
  dst = path_index[1]

  att_d = att[0, :, :OUT]
  att_s = att[0, :, OUT:]
  eye = jnp.eye(HEADS, dtype=jnp.float32)
  Bd = jnp.einsum("ho,hk->hok", att_d, eye).reshape(HO, HEADS)
  Bs = jnp.einsum("ho,hk->hok", att_s, eye).reshape(HO, HEADS)
  B = jnp.concatenate([Bd, Bs], axis=1)

  ht3, scal, c8 = _project(x, W, B)
  ht = ht3.reshape(2 * N, EMB)

  scalt = jnp.pad(scal, ((0, NP - N), (0, 0))).T.reshape(-1)
  c4 = c8[0, :HEADS] + c8[0, HEADS:]
  c4 = jnp.where(c4 > 0, c4, 0.2 * c4)
  c16 = jnp.pad(c4, (0, 12))

  a_flat = _attn_kernel(scalt, c16, src, dst)
  agg = _msg_kernel(ht, a_flat, src, dst)

  out = _head_mean(agg.reshape(2, NP, EMB), bias.reshape(1, OUT))
  a = a_flat.reshape(HEADS, E).T
  return out, a


# speedup vs baseline: 95.7646x; 1.0022x over previous
import jax
import jax.numpy as jnp
from jax import lax
from jax.experimental import pallas as pl
from jax.experimental.pallas import tpu as pltpu
from jax.experimental.pallas import tpu_sc as plsc

N = 10000
NP = 10240
E = 320000
EMB = 128
HEADS = 4
OUT = 64
HO = HEADS * OUT

EPT = E // 16
CHUNK = 80
NCHUNK = EPT // CHUNK
STRIPE = NP // 16
ABLK = 2000
NABLK = EPT // ABLK
MBLK = 800
NCHB = MBLK // CHUNK
NMBLK = EPT // MBLK


def _proj_body(x_ref, w_ref, b_ref, ht_ref, scal_ref, c8_ref):
  i = pl.program_id(0)
  bn = x_ref.shape[0]
  hblk = jnp.dot(x_ref[...], w_ref[...], preferred_element_type=jnp.float32)
  ht_ref[...] = hblk.reshape(bn, 2, EMB).transpose(1, 0, 2)
  sc = jnp.dot(hblk, b_ref[...], preferred_element_type=jnp.float32)
  scal_ref[...] = sc
  m = jnp.max(sc, axis=0, keepdims=True)

  @pl.when(i == 0)
  def _():
    c8_ref[...] = m

  @pl.when(i > 0)
  def _():
    c8_ref[...] = jnp.maximum(c8_ref[...], m)


def _project(x, W, B):
  bn = 1000
  return pl.pallas_call(
      _proj_body,
      grid=(N // bn,),
      in_specs=[
          pl.BlockSpec((bn, EMB), lambda i: (i, 0)),
          pl.BlockSpec((EMB, HO), lambda i: (0, 0)),
          pl.BlockSpec((HO, 8), lambda i: (0, 0)),
      ],
      out_specs=[
          pl.BlockSpec((2, bn, EMB), lambda i: (0, i, 0)),
          pl.BlockSpec((bn, 8), lambda i: (i, 0)),
          pl.BlockSpec((1, 8), lambda i: (0, 0)),
      ],
      out_shape=[
          jax.ShapeDtypeStruct((2, N, EMB), jnp.float32),
          jax.ShapeDtypeStruct((N, 8), jnp.float32),
          jax.ShapeDtypeStruct((1, 8), jnp.float32),
      ],
  )(x, W, B)


def _attn_body(scalt_h, c16_h, src_h, dst_h, a_h,
               tblD, tblS, tblDen, srcv, dstv, a_st0, a_st1,
               tmp, acc, c_v, bsem, sp_den, sp_comb):
  a_sts = (a_st0, a_st1)
  cid = lax.axis_index("c")
  sid = lax.axis_index("s")
  ebase = sid * EPT
  zero16 = jnp.zeros((16,), jnp.float32)

  pltpu.sync_copy(c16_h, c_v)
  cb = [plsc.load_gather(c_v, [jnp.full((16,), 2 * cid + hd, jnp.int32)])
        for hd in range(2)]
  for hd in range(2):
    pltpu.sync_copy(scalt_h.at[pl.ds((2 * cid + hd) * NP, NP)], tblD.at[hd])
    pltpu.sync_copy(scalt_h.at[pl.ds((4 + 2 * cid + hd) * NP, NP)],
                    tblS.at[hd])

  def zden(j, _):
    tblDen[0, pl.ds(j * 16, 16)] = zero16
    tblDen[1, pl.ds(j * 16, 16)] = zero16
    return 0
  lax.fori_loop(0, NP // 16, zden, 0)

  def p1(k, _):
    e0 = ebase + k * ABLK
    descs = [(src_h.at[pl.ds(e0, ABLK)], srcv),
             (dst_h.at[pl.ds(e0, ABLK)], dstv)]
    for sref, dref in descs:
      pltpu.async_copy(sref, dref, bsem)
    for sref, dref in descs:
      pltpu.make_async_copy(sref, dref, bsem).wait()

    @plsc.parallel_loop(0, ABLK // 16, unroll=5)
    def vec(v):
      sv = srcv[pl.ds(v * 16, 16)]
      dv = dstv[pl.ds(v * 16, 16)]
      for hd in range(2):
        hs = jnp.full((16,), hd, jnp.int32)
        s = plsc.load_gather(tblD, [hs, dv]) + plsc.load_gather(tblS, [hs, sv])
        al = jnp.where(s > 0, s, 0.2 * s)
        ex = jnp.exp(al - cb[hd])
        plsc.addupdate_scatter(tblDen, [hs, dv], ex)
    return 0
  lax.fori_loop(0, NABLK, p1, 0)

  for hd in range(2):
    pltpu.sync_copy(tblDen.at[hd],
                    sp_den.at[pl.ds((sid * 2 + hd) * NP, NP)])
  plsc.subcore_barrier()

  def czero(j, _):
    acc[0, pl.ds(j * 16, 16)] = zero16
    acc[1, pl.ds(j * 16, 16)] = zero16
    return 0
  lax.fori_loop(0, STRIPE // 16, czero, 0)

  def comb(t, _):
    for hd in range(2):
      pltpu.sync_copy(
          sp_den.at[pl.ds((t * 2 + hd) * NP + sid * STRIPE, STRIPE)], tmp)

      def addv(j, _):
        acc[hd, pl.ds(j * 16, 16)] = (acc[hd, pl.ds(j * 16, 16)]
                                      + tmp[pl.ds(j * 16, 16)])
        return 0
      lax.fori_loop(0, STRIPE // 16, addv, 0)
    return 0
  lax.fori_loop(0, 16, comb, 0)

  for hd in range(2):
    pltpu.sync_copy(acc.at[hd],
                    sp_comb.at[pl.ds(hd * NP + sid * STRIPE, STRIPE)])
  plsc.subcore_barrier()
  for hd in range(2):
    pltpu.sync_copy(sp_comb.at[pl.ds(hd * NP, NP)], tblDen.at[hd])

  def p2(k, _):
    e0 = ebase + k * ABLK
    descs = [(src_h.at[pl.ds(e0, ABLK)], srcv),
             (dst_h.at[pl.ds(e0, ABLK)], dstv)]
    for sref, dref in descs:
      pltpu.async_copy(sref, dref, bsem)
    for sref, dref in descs:
      pltpu.make_async_copy(sref, dref, bsem).wait()

    @plsc.parallel_loop(0, ABLK // 16, unroll=5)
    def vec(v):
      sv = srcv[pl.ds(v * 16, 16)]
      dv = dstv[pl.ds(v * 16, 16)]
      for hd in range(2):
        hs = jnp.full((16,), hd, jnp.int32)
        s = plsc.load_gather(tblD, [hs, dv]) + plsc.load_gather(tblS, [hs, sv])
        al = jnp.where(s > 0, s, 0.2 * s)
        ex = jnp.exp(al - cb[hd])
        den = plsc.load_gather(tblDen, [hs, dv])
        a_sts[hd][pl.ds(v * 16, 16)] = ex / (den + 1e-16)

    for hd in range(2):
      pltpu.sync_copy(a_sts[hd],
                      a_h.at[pl.ds((2 * cid + hd) * E + e0, ABLK)])
    return 0
  lax.fori_loop(0, NABLK, p2, 0)


def _attn_kernel(scalt, c16, src, dst):
  mesh = plsc.VectorSubcoreMesh(core_axis_name="c", subcore_axis_name="s")
  k = pl.kernel(
      _attn_body,
      out_type=jax.ShapeDtypeStruct((HEADS * E,), jnp.float32),
      mesh=mesh,
      compiler_params=pltpu.CompilerParams(needs_layout_passes=False),
      scratch_types=[
          pltpu.VMEM((2, NP), jnp.float32),
          pltpu.VMEM((2, NP), jnp.float32),
          pltpu.VMEM((2, NP), jnp.float32),
          pltpu.VMEM((ABLK,), jnp.int32),
          pltpu.VMEM((ABLK,), jnp.int32),
          pltpu.VMEM((ABLK,), jnp.float32),
          pltpu.VMEM((ABLK,), jnp.float32),
          pltpu.VMEM((STRIPE,), jnp.float32),
          pltpu.VMEM((2, STRIPE), jnp.float32),
          pltpu.VMEM((16,), jnp.float32),
          pltpu.SemaphoreType.DMA,
          pltpu.VMEM_SHARED((16 * 2 * NP,), jnp.float32),
          pltpu.VMEM_SHARED((2 * NP,), jnp.float32),
      ],
  )
  return k(scalt, c16, src, dst)


def _msg_body(ht_h, a_h, src_h, dst_h, agg_h,
              rows0, rows1, out0, out1, srcv, sadj, dstv2, a_st0, a_st1,
              gsem0, gsem1, ssem0, ssem1, bsem, sp_agg):
  a_sts = (a_st0, a_st1)
  rows = (rows0, rows1)
  out = (out0, out1)
  gsem = (gsem0, gsem1)
  ssem = (ssem0, ssem1)
  cid = lax.axis_index("c")
  sid = lax.axis_index("s")
  ebase = sid * EPT
  zero16 = jnp.zeros((16,), jnp.float32)

  def zrow(i2, _):
    for j in range(8):
      rows0[i2, pl.ds(j * 16, 16)] = zero16
    return 0
  lax.fori_loop(0, CHUNK, zrow, 0)

  def zsp(b, _):
    pltpu.sync_copy(rows0, sp_agg.at[pl.ds(sid * STRIPE + b * CHUNK, CHUNK)])
    return 0
  lax.fori_loop(0, STRIPE // CHUNK, zsp, 0)
  plsc.subcore_barrier()

  def blk(kb, _):
    e0 = ebase + kb * MBLK
    descs = [(src_h.at[pl.ds(e0, MBLK)], srcv)]
    for jj in range(NCHB):
      descs.append((dst_h.at[pl.ds(e0 + jj * CHUNK, CHUNK)], dstv2.at[jj]))
    for hd in range(2):
      descs.append((a_h.at[pl.ds((2 * cid + hd) * E + e0, MBLK)],
                    a_sts[hd]))
    for sref, dref in descs:
      pltpu.async_copy(sref, dref, bsem)
    for sref, dref in descs:
      pltpu.make_async_copy(sref, dref, bsem).wait()

    @plsc.parallel_loop(0, MBLK // 16, unroll=5)
    def vec(v):
      sadj[pl.ds(v * 16, 16)] = srcv[pl.ds(v * 16, 16)] + cid * N

    pltpu.async_copy(ht_h.at[sadj.at[pl.ds(0, CHUNK)]], rows0, gsem0)
    pltpu.async_copy(ht_h.at[sadj.at[pl.ds(CHUNK, CHUNK)]], rows1, gsem1)

    def pair(j2, _):
      for b in range(2):
        j = 2 * j2 + b
        pltpu.make_async_copy(
            ht_h.at[sadj.at[pl.ds(j * CHUNK, CHUNK)]], rows[b],
            gsem[b]).wait()

        @pl.when(j2 >= 1)
        def _():
          pltpu.make_async_copy(out[b], sp_agg.at[dstv2.at[j - 2]],
                                ssem[b]).wait()

        @plsc.parallel_loop(0, CHUNK, unroll=8)
        def scale(i2):
          i2v = jnp.full((16,), j * CHUNK + i2, jnp.int32)
          for hd in range(2):
            ab = plsc.load_gather(a_sts[hd], [i2v])
            for j4 in range(4):
              col = hd * 64 + j4 * 16
              out[b][i2, pl.ds(col, 16)] = rows[b][i2, pl.ds(col, 16)] * ab

        pltpu.async_copy(out[b], sp_agg.at[dstv2.at[j]], ssem[b], add=True)

        @pl.when(j + 2 < NCHB)
        def _():
          pltpu.async_copy(ht_h.at[sadj.at[pl.ds((j + 2) * CHUNK, CHUNK)]],
                           rows[b], gsem[b])
      return 0
    lax.fori_loop(0, NCHB // 2, pair, 0)

    for b in range(2):
      pltpu.make_async_copy(out[b], sp_agg.at[dstv2.at[NCHB - 2 + b]],
                            ssem[b]).wait()
    return 0
  lax.fori_loop(0, NMBLK, blk, 0)

  plsc.subcore_barrier()
  pltpu.sync_copy(
      sp_agg.at[pl.ds(sid * STRIPE, STRIPE)],
      agg_h.at[pl.ds(cid * NP + sid * STRIPE, STRIPE)])


def _msg_kernel(ht, a_flat, src, dst):
  mesh = plsc.VectorSubcoreMesh(core_axis_name="c", subcore_axis_name="s")
  k = pl.kernel(
      _msg_body,
      out_type=jax.ShapeDtypeStruct((2 * NP, EMB), jnp.float32),
      mesh=mesh,
      compiler_params=pltpu.CompilerParams(needs_layout_passes=False),
      scratch_types=[
          pltpu.VMEM((CHUNK, EMB), jnp.float32),
          pltpu.VMEM((CHUNK, EMB), jnp.float32),
          pltpu.VMEM((CHUNK, EMB), jnp.float32),
          pltpu.VMEM((CHUNK, EMB), jnp.float32),
          pltpu.VMEM((MBLK,), jnp.int32),
          pltpu.VMEM((MBLK,), jnp.int32),
          pltpu.VMEM((NCHB, CHUNK), jnp.int32),
          pltpu.VMEM((MBLK,), jnp.float32),
          pltpu.VMEM((MBLK,), jnp.float32),
          pltpu.SemaphoreType.DMA,
          pltpu.SemaphoreType.DMA,
          pltpu.SemaphoreType.DMA,
          pltpu.SemaphoreType.DMA,
          pltpu.SemaphoreType.DMA,
          pltpu.VMEM_SHARED((NP, EMB), jnp.float32),
      ],
  )
  return k(ht, a_flat, src, dst)


def _mean_body(agg_ref, bias_ref, out_ref):
  s = (agg_ref[0, :, 0:OUT] + agg_ref[0, :, OUT:EMB]
       + agg_ref[1, :, 0:OUT] + agg_ref[1, :, OUT:EMB])
  out_ref[...] = s * 0.25 + bias_ref[...]


def _head_mean(agg, bias2d):
  bn = 512
  return pl.pallas_call(
      _mean_body,
      grid=(NP // bn,),
      in_specs=[
          pl.BlockSpec((2, bn, EMB), lambda i: (0, i, 0)),
          pl.BlockSpec((1, OUT), lambda i: (0, 0)),
      ],
      out_specs=pl.BlockSpec((bn, OUT), lambda i: (i, 0)),
      out_shape=jax.ShapeDtypeStruct((N, OUT), jnp.float32),
  )(agg, bias2d)


def kernel(x, path_index, W, att, bias):
  src = path_index[0]
  dst = path_index[1]

  att_d = att[0, :, :OUT]
  att_s = att[0, :, OUT:]
  eye = jnp.eye(HEADS, dtype=jnp.float32)
  Bd = jnp.einsum("ho,hk->hok", att_d, eye).reshape(HO, HEADS)
  Bs = jnp.einsum("ho,hk->hok", att_s, eye).reshape(HO, HEADS)
  B = jnp.concatenate([Bd, Bs], axis=1)

  ht3, scal, c8 = _project(x, W, B)
  ht = ht3.reshape(2 * N, EMB)

  scalt = jnp.pad(scal, ((0, NP - N), (0, 0))).T.reshape(-1)
  c4 = c8[0, :HEADS] + c8[0, HEADS:]
  c4 = jnp.where(c4 > 0, c4, 0.2 * c4)
  c16 = jnp.pad(c4, (0, 12))

  a_flat = _attn_kernel(scalt, c16, src, dst)
  agg = _msg_kernel(ht, a_flat, src, dst)

  out = _head_mean(agg.reshape(2, NP, EMB), bias.reshape(1, OUT))
  a = a_flat.reshape(HEADS, E).T
  return out, a
